# baseline (speedup 1.0000x reference)
import jax
import jax.numpy as jnp
from jax import lax
from jax.experimental import pallas as pl
from jax.experimental.pallas import tpu as pltpu
from jax.experimental.pallas import tpu_sc as plsc

_N_COARSE = 40962
_N_NEW = 122880
_D = 128
_LANES = 16
_NC, _NS = 2, 16
_NW = _NC * _NS

_ROWS_W = _N_NEW // _NW
_B = 96
_NCH = _ROWS_W // _B
_DEPTH = 4
_NOB = 2
_CPY_W = _N_COARSE // _NW
_CB = 64
_NCPY = _CPY_W // _CB
_CPY_REM = _N_COARSE - _CPY_W * _NW


def _body(x, i0, i1, out, *refs):
    gas = refs[0:_DEPTH]
    gbs = refs[_DEPTH:2 * _DEPTH]
    obs = refs[2 * _DEPTH:2 * _DEPTH + _NOB]
    i0v, i1v = refs[2 * _DEPTH + _NOB:2 * _DEPTH + _NOB + 2]
    sems_base = 2 * _DEPTH + _NOB + 2
    semg = refs[sems_base:sems_base + _DEPTH]
    sems = refs[sems_base + _DEPTH:sems_base + _DEPTH + _NOB]
    semcl = refs[sems_base + _DEPTH + _NOB:sems_base + _DEPTH + _NOB + 2]
    semcs = refs[sems_base + _DEPTH + _NOB + 2:sems_base + _DEPTH + _NOB + 4]
    cid = lax.axis_index("c")
    sid = lax.axis_index("s")
    wid = sid * _NC + cid

    pltpu.sync_copy(i0.at[pl.ds(wid * _ROWS_W, _ROWS_W)], i0v)
    pltpu.sync_copy(i1.at[pl.ds(wid * _ROWS_W, _ROWS_W)], i1v)

    def start_gather(c, i):
        sl = pl.ds(c * _B, _B)
        pltpu.async_copy(x.at[i0v.at[sl]], gas[i], semg[i])
        pltpu.async_copy(x.at[i1v.at[sl]], gbs[i], semg[i])

    def wait_gather(i):
        pltpu.make_async_copy(x.at[pl.ds(0, _B)], gas[i], semg[i]).wait()
        pltpu.make_async_copy(x.at[pl.ds(0, _B)], gbs[i], semg[i]).wait()

    def start_store(c, j):
        base = _N_COARSE + wid * _ROWS_W + c * _B
        pltpu.async_copy(obs[j], out.at[pl.ds(base, _B)], sems[j])

    def wait_store(j):
        pltpu.make_async_copy(obs[j], out.at[pl.ds(0, _B)], sems[j]).wait()

    def avg(i, j):
        a = gas[i]
        b = gbs[i]
        o = obs[j]

        @plsc.parallel_loop(0, _B, step=1, unroll=4)
        def _rows(row):
            for v in range(_D // _LANES):
                sl = pl.ds(v * _LANES, _LANES)
                o[row, sl] = (a[row, sl] + b[row, sl]) * 0.5

    for c in range(_DEPTH):
        start_gather(c, c)

    def cload(t, j):
        pltpu.async_copy(x.at[pl.ds(wid * _CPY_W + t * _CB, _CB)],
                         obs[j].at[pl.ds(0, _CB)], semcl[j])

    def cload_wait(j):
        pltpu.make_async_copy(x.at[pl.ds(0, _CB)],
                              obs[j].at[pl.ds(0, _CB)], semcl[j]).wait()

    def cstore(t, j):
        pltpu.async_copy(obs[j].at[pl.ds(0, _CB)],
                         out.at[pl.ds(wid * _CPY_W + t * _CB, _CB)], semcs[j])

    def cstore_wait(j):
        pltpu.make_async_copy(obs[j].at[pl.ds(0, _CB)],
                              out.at[pl.ds(0, _CB)], semcs[j]).wait()

    cload(0, 0)
    for t in range(_NCPY):
        j = t & 1
        if t + 1 < _NCPY:
            if t >= 1:
                cstore_wait(1 - j)
            cload(t + 1, 1 - j)
        cload_wait(j)
        cstore(t, j)
    cstore_wait((_NCPY - 1) & 1)
    cstore_wait((_NCPY - 2) & 1)

    @pl.when(wid == _NW - 1)
    def _rem():
        pltpu.sync_copy(x.at[pl.ds(_NW * _CPY_W, _CPY_REM)],
                        obs[0].at[pl.ds(0, _CPY_REM)])
        pltpu.sync_copy(obs[0].at[pl.ds(0, _CPY_REM)],
                        out.at[pl.ds(_NW * _CPY_W, _CPY_REM)])

    def group(p, carry):
        for k in range(_DEPTH):
            c = _DEPTH * p + k
            i = k
            j = k % _NOB
            wait_gather(i)

            @pl.when(c >= _NOB)
            def _ws():
                wait_store(j)

            avg(i, j)
            start_store(c, j)

            @pl.when(c + _DEPTH < _NCH)
            def _ng():
                start_gather(c + _DEPTH, i)

        return carry

    lax.fori_loop(0, _NCH // _DEPTH, group, 0)
    for j in range(_NOB):
        wait_store(j)


@jax.jit
def kernel(x, upsample_index):
    i0 = upsample_index[:, 0]
    i1 = upsample_index[:, 1]
    f = pl.kernel(
        _body,
        out_type=jax.ShapeDtypeStruct((_N_COARSE + _N_NEW, _D), jnp.float32),
        mesh=plsc.VectorSubcoreMesh(
            core_axis_name="c", subcore_axis_name="s",
            num_cores=_NC, num_subcores=_NS,
        ),
        scratch_types=(
            [pltpu.VMEM((_B, _D), jnp.float32) for _ in range(2 * _DEPTH)]
            + [pltpu.VMEM((_B, _D), jnp.float32) for _ in range(_NOB)]
            + [pltpu.VMEM((_ROWS_W,), jnp.int32) for _ in range(2)]
            + [pltpu.SemaphoreType.DMA for _ in range(_DEPTH + _NOB + 4)]
        ),
        compiler_params=pltpu.CompilerParams(use_tc_tiling_on_sc=False),
    )
    return f(x, i0, i1)


# speedup vs baseline: 1.0402x; 1.0402x over previous
import jax
import jax.numpy as jnp
from jax import lax
from jax.experimental import pallas as pl
from jax.experimental.pallas import tpu as pltpu
from jax.experimental.pallas import tpu_sc as plsc

_N_COARSE = 40962
_N_NEW = 122880
_D = 128
_LANES = 16
_NC, _NS = 2, 16
_NW = _NC * _NS

_ROWS_W = _N_NEW // _NW
_B = 160
_NCH = _ROWS_W // _B
_DEPTH = 2
_NOB = 2
_CPY_W = _N_COARSE // _NW
_CB = 128
_NCPY = _CPY_W // _CB
_CPY_REM = _N_COARSE - _CPY_W * _NW


def _body(x, i0, i1, out, *refs):
    gas = refs[0:_DEPTH]
    gbs = refs[_DEPTH:2 * _DEPTH]
    obs = refs[2 * _DEPTH:2 * _DEPTH + _NOB]
    i0v, i1v = refs[2 * _DEPTH + _NOB:2 * _DEPTH + _NOB + 2]
    sems_base = 2 * _DEPTH + _NOB + 2
    semg = refs[sems_base:sems_base + _DEPTH]
    sems = refs[sems_base + _DEPTH:sems_base + _DEPTH + _NOB]
    semcl = refs[sems_base + _DEPTH + _NOB:sems_base + _DEPTH + _NOB + 2]
    semcs = refs[sems_base + _DEPTH + _NOB + 2:sems_base + _DEPTH + _NOB + 4]
    cid = lax.axis_index("c")
    sid = lax.axis_index("s")
    wid = sid * _NC + cid

    pltpu.sync_copy(i0.at[pl.ds(wid * _ROWS_W, _ROWS_W)], i0v)
    pltpu.sync_copy(i1.at[pl.ds(wid * _ROWS_W, _ROWS_W)], i1v)

    def start_gather(c, i):
        sl = pl.ds(c * _B, _B)
        pltpu.async_copy(x.at[i0v.at[sl]], gas[i], semg[i])
        pltpu.async_copy(x.at[i1v.at[sl]], gbs[i], semg[i])

    def wait_gather(i):
        pltpu.make_async_copy(x.at[pl.ds(0, _B)], gas[i], semg[i]).wait()
        pltpu.make_async_copy(x.at[pl.ds(0, _B)], gbs[i], semg[i]).wait()

    def start_store(c, j):
        base = _N_COARSE + wid * _ROWS_W + c * _B
        pltpu.async_copy(obs[j], out.at[pl.ds(base, _B)], sems[j])

    def wait_store(j):
        pltpu.make_async_copy(obs[j], out.at[pl.ds(0, _B)], sems[j]).wait()

    def avg(i, j):
        a = gas[i]
        b = gbs[i]
        o = obs[j]

        @plsc.parallel_loop(0, _B, step=1, unroll=4)
        def _rows(row):
            for v in range(_D // _LANES):
                sl = pl.ds(v * _LANES, _LANES)
                o[row, sl] = (a[row, sl] + b[row, sl]) * 0.5

    for c in range(_DEPTH):
        start_gather(c, c)

    def cload(t, j):
        pltpu.async_copy(x.at[pl.ds(wid * _CPY_W + t * _CB, _CB)],
                         obs[j].at[pl.ds(0, _CB)], semcl[j])

    def cload_wait(j):
        pltpu.make_async_copy(x.at[pl.ds(0, _CB)],
                              obs[j].at[pl.ds(0, _CB)], semcl[j]).wait()

    def cstore(t, j):
        pltpu.async_copy(obs[j].at[pl.ds(0, _CB)],
                         out.at[pl.ds(wid * _CPY_W + t * _CB, _CB)], semcs[j])

    def cstore_wait(j):
        pltpu.make_async_copy(obs[j].at[pl.ds(0, _CB)],
                              out.at[pl.ds(0, _CB)], semcs[j]).wait()

    cload(0, 0)
    for t in range(_NCPY):
        j = t & 1
        if t + 1 < _NCPY:
            if t >= 1:
                cstore_wait(1 - j)
            cload(t + 1, 1 - j)
        cload_wait(j)
        cstore(t, j)
    cstore_wait((_NCPY - 1) & 1)
    cstore_wait((_NCPY - 2) & 1)

    @pl.when(wid == _NW - 1)
    def _rem():
        pltpu.sync_copy(x.at[pl.ds(_NW * _CPY_W, _CPY_REM)],
                        obs[0].at[pl.ds(0, _CPY_REM)])
        pltpu.sync_copy(obs[0].at[pl.ds(0, _CPY_REM)],
                        out.at[pl.ds(_NW * _CPY_W, _CPY_REM)])

    def group(p, carry):
        for k in range(_DEPTH):
            c = _DEPTH * p + k
            i = k
            j = k % _NOB
            wait_gather(i)

            @pl.when(c >= _NOB)
            def _ws():
                wait_store(j)

            avg(i, j)
            start_store(c, j)

            @pl.when(c + _DEPTH < _NCH)
            def _ng():
                start_gather(c + _DEPTH, i)

        return carry

    lax.fori_loop(0, _NCH // _DEPTH, group, 0)
    for j in range(_NOB):
        wait_store(j)


@jax.jit
def kernel(x, upsample_index):
    i0 = upsample_index[:, 0]
    i1 = upsample_index[:, 1]
    f = pl.kernel(
        _body,
        out_type=jax.ShapeDtypeStruct((_N_COARSE + _N_NEW, _D), jnp.float32),
        mesh=plsc.VectorSubcoreMesh(
            core_axis_name="c", subcore_axis_name="s",
            num_cores=_NC, num_subcores=_NS,
        ),
        scratch_types=(
            [pltpu.VMEM((_B, _D), jnp.float32) for _ in range(2 * _DEPTH)]
            + [pltpu.VMEM((_B, _D), jnp.float32) for _ in range(_NOB)]
            + [pltpu.VMEM((_ROWS_W,), jnp.int32) for _ in range(2)]
            + [pltpu.SemaphoreType.DMA for _ in range(_DEPTH + _NOB + 4)]
        ),
        compiler_params=pltpu.CompilerParams(use_tc_tiling_on_sc=False),
    )
    return f(x, i0, i1)


# speedup vs baseline: 1.0454x; 1.0050x over previous
import jax
import jax.numpy as jnp
from jax import lax
from jax.experimental import pallas as pl
from jax.experimental.pallas import tpu as pltpu
from jax.experimental.pallas import tpu_sc as plsc

_N_COARSE = 40962
_N_NEW = 122880
_D = 128
_LANES = 16
_NC, _NS = 2, 16
_NW = _NC * _NS

_ROWS_W = _N_NEW // _NW
_B = 128
_NCH = _ROWS_W // _B
_NPAIR = _NCH // 2
_CPY_W = _N_COARSE // _NW
_CB = 128
_NCPY = _CPY_W // _CB
_CPY_REM = _N_COARSE - _CPY_W * _NW


def _body(x, i0, i1, out, ga0, ga1, gb0, gb1, ob0, ob1, i0v, i1v,
          semg0, semg1, sems0, sems1, semcl0, semcl1, semcs0, semcs1):
    gas = (ga0, ga1)
    gbs = (gb0, gb1)
    obs = (ob0, ob1)
    semg = (semg0, semg1)
    sems = (sems0, sems1)
    semcl = (semcl0, semcl1)
    semcs = (semcs0, semcs1)
    cid = lax.axis_index("c")
    sid = lax.axis_index("s")
    wid = sid * _NC + cid

    pltpu.sync_copy(i0.at[pl.ds(wid * _ROWS_W, _ROWS_W)], i0v)
    pltpu.sync_copy(i1.at[pl.ds(wid * _ROWS_W, _ROWS_W)], i1v)

    def start_gather(c, i):
        sl = pl.ds(c * _B, _B)
        pltpu.async_copy(x.at[i0v.at[sl]], gas[i], semg[i])
        pltpu.async_copy(x.at[i1v.at[sl]], gbs[i], semg[i])

    def wait_gather(i):
        pltpu.make_async_copy(x.at[pl.ds(0, _B)], gas[i], semg[i]).wait()
        pltpu.make_async_copy(x.at[pl.ds(0, _B)], gbs[i], semg[i]).wait()

    def start_store(c, i):
        base = _N_COARSE + wid * _ROWS_W + c * _B
        pltpu.async_copy(obs[i], out.at[pl.ds(base, _B)], sems[i])

    def wait_store(i):
        pltpu.make_async_copy(obs[i], out.at[pl.ds(0, _B)], sems[i]).wait()

    def avg(i):
        a = gas[i]
        b = gbs[i]
        o = obs[i]

        @plsc.parallel_loop(0, _B, step=1, unroll=4)
        def _rows(row):
            for v in range(_D // _LANES):
                sl = pl.ds(v * _LANES, _LANES)
                o[row, sl] = (a[row, sl] + b[row, sl]) * 0.5

    start_gather(0, 0)
    start_gather(1, 1)

    def cload(t, j):
        pltpu.async_copy(x.at[pl.ds(wid * _CPY_W + t * _CB, _CB)],
                         obs[j], semcl[j])

    def cload_wait(j):
        pltpu.make_async_copy(x.at[pl.ds(0, _CB)], obs[j], semcl[j]).wait()

    def cstore(t, j):
        pltpu.async_copy(obs[j], out.at[pl.ds(wid * _CPY_W + t * _CB, _CB)],
                         semcs[j])

    def cstore_wait(j):
        pltpu.make_async_copy(obs[j], out.at[pl.ds(0, _CB)], semcs[j]).wait()

    cload(0, 0)
    for t in range(_NCPY):
        j = t & 1
        if t + 1 < _NCPY:
            if t >= 1:
                cstore_wait(1 - j)
            cload(t + 1, 1 - j)
        cload_wait(j)
        cstore(t, j)
    cstore_wait((_NCPY - 1) & 1)
    cstore_wait((_NCPY - 2) & 1)

    @pl.when(wid == _NW - 1)
    def _rem():
        pltpu.sync_copy(x.at[pl.ds(_NW * _CPY_W, _CPY_REM)],
                        ob0.at[pl.ds(0, _CPY_REM)])
        pltpu.sync_copy(ob0.at[pl.ds(0, _CPY_REM)],
                        out.at[pl.ds(_NW * _CPY_W, _CPY_REM)])

    def pair(p, carry):
        for i in range(2):
            c = 2 * p + i
            wait_gather(i)

            @pl.when(c >= 2)
            def _ws():
                wait_store(i)

            avg(i)
            start_store(c, i)

            @pl.when(p < _NPAIR - 1)
            def _ng():
                start_gather(c + 2, i)

        return carry

    lax.fori_loop(0, _NPAIR, pair, 0)
    wait_store(0)
    wait_store(1)


@jax.jit
def kernel(x, upsample_index):
    i0 = upsample_index[:, 0]
    i1 = upsample_index[:, 1]
    f = pl.kernel(
        _body,
        out_type=jax.ShapeDtypeStruct((_N_COARSE + _N_NEW, _D), jnp.float32),
        mesh=plsc.VectorSubcoreMesh(
            core_axis_name="c", subcore_axis_name="s",
            num_cores=_NC, num_subcores=_NS,
        ),
        scratch_types=[
            pltpu.VMEM((_B, _D), jnp.float32),
            pltpu.VMEM((_B, _D), jnp.float32),
            pltpu.VMEM((_B, _D), jnp.float32),
            pltpu.VMEM((_B, _D), jnp.float32),
            pltpu.VMEM((_B, _D), jnp.float32),
            pltpu.VMEM((_B, _D), jnp.float32),
            pltpu.VMEM((_ROWS_W,), jnp.int32),
            pltpu.VMEM((_ROWS_W,), jnp.int32),
            pltpu.SemaphoreType.DMA,
            pltpu.SemaphoreType.DMA,
            pltpu.SemaphoreType.DMA,
            pltpu.SemaphoreType.DMA,
            pltpu.SemaphoreType.DMA,
            pltpu.SemaphoreType.DMA,
            pltpu.SemaphoreType.DMA,
            pltpu.SemaphoreType.DMA,
        ],
        compiler_params=pltpu.CompilerParams(use_tc_tiling_on_sc=False),
    )
    return f(x, i0, i1)


# speedup vs baseline: 1.0483x; 1.0028x over previous
import jax
import jax.numpy as jnp
from jax import lax
from jax.experimental import pallas as pl
from jax.experimental.pallas import tpu as pltpu
from jax.experimental.pallas import tpu_sc as plsc

_N_COARSE = 40962
_N_NEW = 122880
_D = 128
_LANES = 16
_NC, _NS = 2, 16
_NW = _NC * _NS

_ROWS_W = _N_NEW // _NW
_B = 128
_NCH = _ROWS_W // _B
_NPAIR = _NCH // 2
_CPY_W = _N_COARSE // _NW
_CB = 128
_NCPY = _CPY_W // _CB
_CPY_REM = _N_COARSE - _CPY_W * _NW


def _body(x, i0, i1, out, ga0, ga1, gb0, gb1, ob0, ob1, rbuf, i0v, i1v,
          semg0, semg1, sems0, sems1, semcl0, semcl1, semcs0, semcs1, semr):
    gas = (ga0, ga1)
    gbs = (gb0, gb1)
    obs = (ob0, ob1)
    semg = (semg0, semg1)
    sems = (sems0, sems1)
    semcl = (semcl0, semcl1)
    semcs = (semcs0, semcs1)
    cid = lax.axis_index("c")
    sid = lax.axis_index("s")
    wid = sid * _NC + cid

    pltpu.async_copy(i0.at[pl.ds(wid * _ROWS_W, _ROWS_W)], i0v, semcl0)
    pltpu.async_copy(i1.at[pl.ds(wid * _ROWS_W, _ROWS_W)], i1v, semcl1)
    pltpu.make_async_copy(i0.at[pl.ds(0, _ROWS_W)], i0v, semcl0).wait()
    pltpu.make_async_copy(i1.at[pl.ds(0, _ROWS_W)], i1v, semcl1).wait()

    def start_gather(c, i):
        sl = pl.ds(c * _B, _B)
        pltpu.async_copy(x.at[i0v.at[sl]], gas[i], semg[i])
        pltpu.async_copy(x.at[i1v.at[sl]], gbs[i], semg[i])

    def wait_gather(i):
        pltpu.make_async_copy(x.at[pl.ds(0, _B)], gas[i], semg[i]).wait()
        pltpu.make_async_copy(x.at[pl.ds(0, _B)], gbs[i], semg[i]).wait()

    def start_store(c, i):
        base = _N_COARSE + wid * _ROWS_W + c * _B
        pltpu.async_copy(obs[i], out.at[pl.ds(base, _B)], sems[i])

    def wait_store(i):
        pltpu.make_async_copy(obs[i], out.at[pl.ds(0, _B)], sems[i]).wait()

    def avg(i):
        a = gas[i]
        b = gbs[i]
        o = obs[i]

        @plsc.parallel_loop(0, _B, step=1, unroll=4)
        def _rows(row):
            for v in range(_D // _LANES):
                sl = pl.ds(v * _LANES, _LANES)
                o[row, sl] = (a[row, sl] + b[row, sl]) * 0.5

    start_gather(0, 0)
    start_gather(1, 1)

    def cload(t, j):
        pltpu.async_copy(x.at[pl.ds(wid * _CPY_W + t * _CB, _CB)],
                         obs[j], semcl[j])

    def cload_wait(j):
        pltpu.make_async_copy(x.at[pl.ds(0, _CB)], obs[j], semcl[j]).wait()

    def cstore(t, j):
        pltpu.async_copy(obs[j], out.at[pl.ds(wid * _CPY_W + t * _CB, _CB)],
                         semcs[j])

    def cstore_wait(j):
        pltpu.make_async_copy(obs[j], out.at[pl.ds(0, _CB)], semcs[j]).wait()

    cload(0, 0)
    for t in range(_NCPY):
        j = t & 1
        if t + 1 < _NCPY:
            if t >= 1:
                cstore_wait(1 - j)
            cload(t + 1, 1 - j)
        cload_wait(j)
        cstore(t, j)
    cstore_wait((_NCPY - 1) & 1)
    cstore_wait((_NCPY - 2) & 1)

    @pl.when(wid == _NW - 1)
    def _rem_load():
        pltpu.async_copy(x.at[pl.ds(_NW * _CPY_W, _CPY_REM)], rbuf, semr)

    def pair(p, carry):
        for i in range(2):
            c = 2 * p + i
            wait_gather(i)

            @pl.when(c >= 2)
            def _ws():
                wait_store(i)

            avg(i)
            start_store(c, i)

            @pl.when(p < _NPAIR - 1)
            def _ng():
                start_gather(c + 2, i)

        return carry

    lax.fori_loop(0, _NPAIR, pair, 0)
    wait_store(0)
    wait_store(1)

    @pl.when(wid == _NW - 1)
    def _rem_store():
        pltpu.make_async_copy(x.at[pl.ds(0, _CPY_REM)], rbuf, semr).wait()
        pltpu.async_copy(rbuf, out.at[pl.ds(_NW * _CPY_W, _CPY_REM)], semr)
        pltpu.make_async_copy(rbuf, out.at[pl.ds(0, _CPY_REM)], semr).wait()


@jax.jit
def kernel(x, upsample_index):
    i0 = upsample_index[:, 0]
    i1 = upsample_index[:, 1]
    f = pl.kernel(
        _body,
        out_type=jax.ShapeDtypeStruct((_N_COARSE + _N_NEW, _D), jnp.float32),
        mesh=plsc.VectorSubcoreMesh(
            core_axis_name="c", subcore_axis_name="s",
            num_cores=_NC, num_subcores=_NS,
        ),
        scratch_types=[
            pltpu.VMEM((_B, _D), jnp.float32),
            pltpu.VMEM((_B, _D), jnp.float32),
            pltpu.VMEM((_B, _D), jnp.float32),
            pltpu.VMEM((_B, _D), jnp.float32),
            pltpu.VMEM((_B, _D), jnp.float32),
            pltpu.VMEM((_B, _D), jnp.float32),
            pltpu.VMEM((_CPY_REM, _D), jnp.float32),
            pltpu.VMEM((_ROWS_W,), jnp.int32),
            pltpu.VMEM((_ROWS_W,), jnp.int32),
            pltpu.SemaphoreType.DMA,
            pltpu.SemaphoreType.DMA,
            pltpu.SemaphoreType.DMA,
            pltpu.SemaphoreType.DMA,
            pltpu.SemaphoreType.DMA,
            pltpu.SemaphoreType.DMA,
            pltpu.SemaphoreType.DMA,
            pltpu.SemaphoreType.DMA,
            pltpu.SemaphoreType.DMA,
        ],
        compiler_params=pltpu.CompilerParams(use_tc_tiling_on_sc=False),
    )
    return f(x, i0, i1)


# speedup vs baseline: 1.0603x; 1.0114x over previous
import jax
import jax.numpy as jnp
from jax import lax
from jax.experimental import pallas as pl
from jax.experimental.pallas import tpu as pltpu
from jax.experimental.pallas import tpu_sc as plsc

_N_COARSE = 40962
_N_NEW = 122880
_D = 128
_LANES = 16
_NC, _NS = 2, 16
_NW = _NC * _NS

_ROWS_W = _N_NEW // _NW
_B = 128
_NCH = _ROWS_W // _B
_NPAIR = _NCH // 2
_CPY_W = _N_COARSE // _NW
_CB = 128
_NCPY = _CPY_W // _CB
_CPY_REM = _N_COARSE - _CPY_W * _NW


def _body(x, ic, out, gab0, gab1, ob0, ob1, rbuf, icv,
          semg0, semg1, sems0, sems1, semcl0, semcl1, semcs0, semcs1, semr):
    gabs = (gab0, gab1)
    obs = (ob0, ob1)
    semg = (semg0, semg1)
    sems = (sems0, sems1)
    semcl = (semcl0, semcl1)
    semcs = (semcs0, semcs1)
    cid = lax.axis_index("c")
    sid = lax.axis_index("s")
    wid = sid * _NC + cid

    pltpu.sync_copy(ic.at[pl.ds(wid * 2 * _ROWS_W, 2 * _ROWS_W)], icv)

    def start_gather(c, i):
        pltpu.async_copy(x.at[icv.at[pl.ds(c * 2 * _B, 2 * _B)]],
                         gabs[i], semg[i])

    def wait_gather(i):
        pltpu.make_async_copy(x.at[pl.ds(0, 2 * _B)], gabs[i], semg[i]).wait()

    def start_store(c, i):
        base = _N_COARSE + wid * _ROWS_W + c * _B
        pltpu.async_copy(obs[i], out.at[pl.ds(base, _B)], sems[i])

    def wait_store(i):
        pltpu.make_async_copy(obs[i], out.at[pl.ds(0, _B)], sems[i]).wait()

    def avg(i):
        g = gabs[i]
        o = obs[i]

        @plsc.parallel_loop(0, _B, step=1, unroll=4)
        def _rows(row):
            for v in range(_D // _LANES):
                sl = pl.ds(v * _LANES, _LANES)
                o[row, sl] = (g[row, sl] + g[_B + row, sl]) * 0.5

    start_gather(0, 0)
    start_gather(1, 1)

    def cload(t, j):
        pltpu.async_copy(x.at[pl.ds(wid * _CPY_W + t * _CB, _CB)],
                         obs[j], semcl[j])

    def cload_wait(j):
        pltpu.make_async_copy(x.at[pl.ds(0, _CB)], obs[j], semcl[j]).wait()

    def cstore(t, j):
        pltpu.async_copy(obs[j], out.at[pl.ds(wid * _CPY_W + t * _CB, _CB)],
                         semcs[j])

    def cstore_wait(j):
        pltpu.make_async_copy(obs[j], out.at[pl.ds(0, _CB)], semcs[j]).wait()

    cload(0, 0)
    for t in range(_NCPY):
        j = t & 1
        if t + 1 < _NCPY:
            if t >= 1:
                cstore_wait(1 - j)
            cload(t + 1, 1 - j)
        cload_wait(j)
        cstore(t, j)
    cstore_wait((_NCPY - 1) & 1)
    cstore_wait((_NCPY - 2) & 1)

    @pl.when(wid == _NW - 1)
    def _rem_load():
        pltpu.async_copy(x.at[pl.ds(_NW * _CPY_W, _CPY_REM)], rbuf, semr)

    def pair(p, carry):
        for i in range(2):
            c = 2 * p + i
            wait_gather(i)

            @pl.when(c >= 2)
            def _ws():
                wait_store(i)

            avg(i)
            start_store(c, i)

            @pl.when(p < _NPAIR - 1)
            def _ng():
                start_gather(c + 2, i)

        return carry

    lax.fori_loop(0, _NPAIR, pair, 0)
    wait_store(0)
    wait_store(1)

    @pl.when(wid == _NW - 1)
    def _rem_store():
        pltpu.make_async_copy(x.at[pl.ds(0, _CPY_REM)], rbuf, semr).wait()
        pltpu.async_copy(rbuf, out.at[pl.ds(_NW * _CPY_W, _CPY_REM)], semr)
        pltpu.make_async_copy(rbuf, out.at[pl.ds(0, _CPY_REM)], semr).wait()


@jax.jit
def kernel(x, upsample_index):
    ic = (upsample_index.reshape(_N_NEW // _B, _B, 2)
          .transpose(0, 2, 1).reshape(-1))
    f = pl.kernel(
        _body,
        out_type=jax.ShapeDtypeStruct((_N_COARSE + _N_NEW, _D), jnp.float32),
        mesh=plsc.VectorSubcoreMesh(
            core_axis_name="c", subcore_axis_name="s",
            num_cores=_NC, num_subcores=_NS,
        ),
        scratch_types=[
            pltpu.VMEM((2 * _B, _D), jnp.float32),
            pltpu.VMEM((2 * _B, _D), jnp.float32),
            pltpu.VMEM((_B, _D), jnp.float32),
            pltpu.VMEM((_B, _D), jnp.float32),
            pltpu.VMEM((_CPY_REM, _D), jnp.float32),
            pltpu.VMEM((2 * _ROWS_W,), jnp.int32),
            pltpu.SemaphoreType.DMA,
            pltpu.SemaphoreType.DMA,
            pltpu.SemaphoreType.DMA,
            pltpu.SemaphoreType.DMA,
            pltpu.SemaphoreType.DMA,
            pltpu.SemaphoreType.DMA,
            pltpu.SemaphoreType.DMA,
            pltpu.SemaphoreType.DMA,
            pltpu.SemaphoreType.DMA,
        ],
        compiler_params=pltpu.CompilerParams(use_tc_tiling_on_sc=False),
    )
    return f(x, ic)
